# baseline (speedup 1.0000x reference)
import functools

import jax
import jax.numpy as jnp
from jax import lax
from jax.experimental import pallas as pl
from jax.experimental.pallas import tpu as pltpu
from jax.experimental.pallas import tpu_sc as plsc

_D = 128
_B = 16384
_NC = 2
_NS = 16
_NW = _NC * _NS
_BPW = _B // _NW
_H = 256
_HG = _BPW - _H
_NJ = _D // 16

_mesh = plsc.VectorSubcoreMesh(core_axis_name="c", subcore_axis_name="s")


@functools.partial(
    pl.kernel,
    mesh=_mesh,
    out_type=jax.ShapeDtypeStruct((1, _B, _D), jnp.float32),
    scratch_types=[
        pltpu.VMEM_SHARED((_NS, 2, _D), jnp.float32),
        pltpu.VMEM((2 * _D,), jnp.float32),
        pltpu.VMEM((_BPW,), jnp.int32),
        pltpu.VMEM((_BPW, _D), jnp.float32),
    ] + [pltpu.SemaphoreType.DMA] * 9,
)
def _select_kernel(table_hbm, tflat_hbm, idx_hbm, out_hbm,
                   table_s, table_v, flags_v, rows_v,
                   sem_t, sem_v, sem_f, g0, g1, o0, o1, o2, o3):
    cid = lax.axis_index("c")
    sid = lax.axis_index("s")
    wid = sid * _NC + cid
    base = wid * _BPW
    out2d = out_hbm.at[0]
    cp_t = pltpu.async_copy(table_hbm, table_s.at[sid], sem_t)
    cp_v = pltpu.async_copy(tflat_hbm, table_v, sem_v)
    cp_f = pltpu.async_copy(idx_hbm.at[pl.ds(base, _BPW)], flags_v, sem_f)
    cp_t.wait()
    cp_f.wait()
    hg = _HG // 2
    gath0 = pltpu.async_copy(
        table_s.at[sid].at[flags_v.at[pl.ds(_H, hg)]],
        rows_v.at[pl.ds(_H, hg)], g0)
    gath1 = pltpu.async_copy(
        table_s.at[sid].at[flags_v.at[pl.ds(_H + hg, hg)]],
        rows_v.at[pl.ds(_H + hg, hg)], g1)
    cp_v.wait()
    pres = [table_v[pl.ds(16 * j, 16)] for j in range(_NJ)]
    diff = [table_v[pl.ds(_D + 16 * j, 16)] - pres[j] for j in range(_NJ)]
    lane = [jnp.full((16, 1), l, jnp.int32) for l in range(16)]
    dnums = lax.GatherDimensionNumbers(
        offset_dims=(), collapsed_slice_dims=(0,), start_index_map=(0,))

    def body(g, carry):
        fv = flags_v[pl.ds(g * 16, 16)]
        rbase = g * 16
        for l in range(16):
            bl = lax.gather(fv, lane[l], dnums, (1,),
                            mode=lax.GatherScatterMode.PROMISE_IN_BOUNDS)
            f = bl.astype(jnp.float32)
            for j in range(_NJ):
                rows_v[rbase + l, pl.ds(16 * j, 16)] = pres[j] + f * diff[j]
        return carry

    hc = _H // 2
    lax.fori_loop(0, hc // 16, body, 0)
    cpo0 = pltpu.async_copy(rows_v.at[pl.ds(0, hc)],
                            out2d.at[pl.ds(base, hc)], o0)
    gath0.wait()
    cpo1 = pltpu.async_copy(rows_v.at[pl.ds(_H, hg)],
                            out2d.at[pl.ds(base + _H, hg)], o1)
    lax.fori_loop(hc // 16, _H // 16, body, 0)
    cpo2 = pltpu.async_copy(rows_v.at[pl.ds(hc, hc)],
                            out2d.at[pl.ds(base + hc, hc)], o2)
    gath1.wait()
    cpo3 = pltpu.async_copy(rows_v.at[pl.ds(_H + hg, hg)],
                            out2d.at[pl.ds(base + _H + hg, hg)], o3)
    cpo0.wait()
    cpo1.wait()
    cpo2.wait()
    cpo3.wait()


def kernel(is_absent, present_goal_vector, absent_goal_vector):
    table = jnp.stack([present_goal_vector, absent_goal_vector])
    idx = is_absent.astype(jnp.int32)
    return _select_kernel(table, table.reshape(-1), idx)


# speedup vs baseline: 1.0308x; 1.0308x over previous
import functools

import jax
import jax.numpy as jnp
from jax import lax
from jax.experimental import pallas as pl
from jax.experimental.pallas import tpu as pltpu
from jax.experimental.pallas import tpu_sc as plsc

_D = 128
_B = 16384
_NC = 2
_NS = 16
_NW = _NC * _NS
_BPW = _B // _NW
_NCH = 8
_CH = _BPW // _NCH

_mesh = plsc.VectorSubcoreMesh(core_axis_name="c", subcore_axis_name="s")


@functools.partial(
    pl.kernel,
    mesh=_mesh,
    out_type=jax.ShapeDtypeStruct((1, _B, _D), jnp.float32),
    scratch_types=[
        pltpu.VMEM_SHARED((_NS, 2, _D), jnp.float32),
        pltpu.VMEM((_BPW,), jnp.int32),
        pltpu.VMEM((_BPW, _D), jnp.float32),
    ] + [pltpu.SemaphoreType.DMA] * 19,
)
def _gather_kernel(pres_hbm, abs_hbm, idx_hbm, out_hbm,
                   table_s, flags_v, rows_v, sem_p, sem_a, sem_o, *ksem):
    cid = lax.axis_index("c")
    sid = lax.axis_index("s")
    wid = sid * _NC + cid
    base = wid * _BPW
    out2d = out_hbm.at[0]
    isem = list(ksem[:_NCH])
    gsem = list(ksem[_NCH:])
    cp_p = pltpu.async_copy(pres_hbm, table_s.at[sid].at[0], sem_p)
    cp_a = pltpu.async_copy(abs_hbm, table_s.at[sid].at[1], sem_a)
    icps = [pltpu.async_copy(idx_hbm.at[pl.ds(base + k * _CH, _CH)],
                             flags_v.at[pl.ds(k * _CH, _CH)], isem[k])
            for k in range(_NCH)]
    cp_p.wait()
    cp_a.wait()
    gaths = []
    for k in range(_NCH):
        icps[k].wait()
        gaths.append(pltpu.async_copy(
            table_s.at[sid].at[flags_v.at[pl.ds(k * _CH, _CH)]],
            rows_v.at[pl.ds(k * _CH, _CH)], gsem[k]))
    outs = []
    for k in range(_NCH):
        gaths[k].wait()
        outs.append(pltpu.async_copy(
            rows_v.at[pl.ds(k * _CH, _CH)],
            out2d.at[pl.ds(base + k * _CH, _CH)], sem_o))
    for o in outs:
        o.wait()


def kernel(is_absent, present_goal_vector, absent_goal_vector):
    idx = is_absent.astype(jnp.int32)
    return _gather_kernel(present_goal_vector, absent_goal_vector, idx)
